# baseline (speedup 1.0000x reference)
import functools

import jax
import jax.numpy as jnp
from jax import lax
from jax.experimental import pallas as pl
from jax.experimental.pallas import tpu as pltpu
from jax.experimental.pallas import tpu_sc as plsc

_BI = 16

_NC = 2
_NS = 16
_L = 16


def _tc_kernel(q_ref, t_ref, o_ref):
    S = q_ref.shape[1]
    D = q_ref.shape[2]
    e = q_ref[0] + t_ref[...]
    o_ref[0] = jnp.broadcast_to(e[None], (_BI, S, D))


def _tc_call(q, table, S, D):
    B = q.shape[0]
    return pl.pallas_call(
        _tc_kernel,
        grid=(S // _BI,),
        in_specs=[
            pl.BlockSpec((B, S, D), lambda i: (0, 0, 0)),
            pl.BlockSpec(table.shape, lambda i: (0, 0)),
        ],
        out_specs=pl.BlockSpec((B, _BI, S, D), lambda i: (0, i, 0, 0)),
        out_shape=jax.ShapeDtypeStruct((B, S, S, D), q.dtype),
    )(q, table)


def _sc_body(S, D, k_hbm, t_hbm, o_hbm, kv, tv, ev, ef, esh, sem):
    rows = S // _NS
    ipw = S // (_NC * _NS)
    cid = lax.axis_index("c")
    sid = lax.axis_index("s")

    r0 = sid * rows
    pltpu.sync_copy(k_hbm.at[0, pl.ds(r0, rows)], kv)
    pltpu.sync_copy(t_hbm.at[pl.ds(r0, rows)], tv)
    for r in range(rows):
        for c in range(D // _L):
            ev[r, pl.ds(c * _L, _L)] = (
                kv[r, pl.ds(c * _L, _L)] + tv[r, pl.ds(c * _L, _L)]
            )
    pltpu.sync_copy(ev, esh.at[pl.ds(r0, rows)])
    plsc.subcore_barrier()

    pltpu.sync_copy(esh, ef)

    base = (cid * _NS + sid) * ipw
    copies = [
        pltpu.async_copy(ef, o_hbm.at[0, base + i], sem) for i in range(ipw)
    ]
    for c in copies:
        c.wait()


def _sc_call(k, table, S, D):
    mesh = plsc.VectorSubcoreMesh(
        core_axis_name="c", subcore_axis_name="s",
        num_cores=_NC, num_subcores=_NS,
    )
    rows = S // _NS
    run = pl.kernel(
        functools.partial(_sc_body, S, D),
        out_type=jax.ShapeDtypeStruct((1, S, S, D), k.dtype),
        mesh=mesh,
        scratch_types=[
            pltpu.VMEM((rows, D), jnp.float32),
            pltpu.VMEM((rows, D), jnp.float32),
            pltpu.VMEM((rows, D), jnp.float32),
            pltpu.VMEM((S, D), jnp.float32),
            pltpu.VMEM_SHARED((S, D), jnp.float32),
            pltpu.SemaphoreType.DMA,
        ],
    )
    return run(k, table)


def kernel(q, k, table):
    B, S, D = q.shape
    t = lax.slice(table, (1, 0), (S + 1, D))
    out1 = _sc_call(q, t, S, D)
    out2 = _sc_call(k, t, S, D)
    return (out1, out2)


# speedup vs baseline: 1.4275x; 1.4275x over previous
import jax
import jax.numpy as jnp
from jax.experimental import pallas as pl

_BI = 16


def _bcast_kernel(q_ref, k_ref, t_ref, o1_ref, o2_ref):
    S = q_ref.shape[1]
    D = q_ref.shape[2]
    t = t_ref[pl.ds(1, S), :]
    e1 = q_ref[0] + t
    e2 = k_ref[0] + t
    o1_ref[0] = jnp.broadcast_to(e1[None], (_BI, S, D))
    o2_ref[0] = jnp.broadcast_to(e2[None], (_BI, S, D))


def kernel(q, k, table):
    B, S, D = q.shape
    grid = (S // _BI,)
    out_shape = (
        jax.ShapeDtypeStruct((B, S, S, D), q.dtype),
        jax.ShapeDtypeStruct((B, S, S, D), q.dtype),
    )
    return pl.pallas_call(
        _bcast_kernel,
        grid=grid,
        in_specs=[
            pl.BlockSpec((B, S, D), lambda i: (0, 0, 0)),
            pl.BlockSpec((B, S, D), lambda i: (0, 0, 0)),
            pl.BlockSpec(table.shape, lambda i: (0, 0)),
        ],
        out_specs=(
            pl.BlockSpec((B, _BI, S, D), lambda i: (0, i, 0, 0)),
            pl.BlockSpec((B, _BI, S, D), lambda i: (0, i, 0, 0)),
        ),
        out_shape=out_shape,
    )(q, k, table)
